# Initial kernel scaffold; baseline (speedup 1.0000x reference)
import math

import jax
import jax.numpy as jnp
from jax import lax
from jax.experimental import pallas as pl
from jax.experimental.pallas import tpu as pltpu
from jax.experimental.pallas import tpu_sc as plsc

N = 10000
E = 160000
D = 256
HALF = 128
R = 8
NB = 4
BN_EPS = 1e-5

NODE_BLK = 400
EDGE_BLK = 640
NW = 32
EPW = E // NW
CHUNK = 128
EPT = E // 16
RPT = N // 16

_f32 = jnp.float32


def _node_proj_body(wc_ref, h_ref, aw_ref, ab_ref, bw_ref, bb_ref, ew_ref,
                    eb_ref, wt_ref, ah_o, bh_o, eh_o, hall_o):
    hb = h_ref[...]
    ah_o[...] = jnp.dot(hb, aw_ref[...], preferred_element_type=_f32) + ab_ref[...]
    bh_o[...] = jnp.dot(hb, bw_ref[...], preferred_element_type=_f32) + bb_ref[...]
    eh_o[...] = jnp.dot(hb, ew_ref[...], preferred_element_type=_f32) + eb_ref[...]
    base = [jnp.dot(hb, wt_ref[b], preferred_element_type=_f32) for b in range(NB)]
    for r in range(R):
        acc = base[0] * wc_ref[0, r * NB]
        for b in range(1, NB):
            acc = acc + base[b] * wc_ref[0, r * NB + b]
        hall_o[r] = acc


def _node_proj(h, A_w, A_b, B_w, B_b, E_w, E_b, weight, w_comp):
    nblk = N // NODE_BLK
    vec_spec = pl.BlockSpec((1, D), lambda i: (0, 0))
    mat_spec = pl.BlockSpec((D, D), lambda i: (0, 0))
    out = pl.pallas_call(
        _node_proj_body,
        grid=(nblk,),
        in_specs=[
            pl.BlockSpec(memory_space=pltpu.SMEM),
            pl.BlockSpec((NODE_BLK, D), lambda i: (i, 0)),
            mat_spec, vec_spec, mat_spec, vec_spec, mat_spec, vec_spec,
            pl.BlockSpec((NB, D, D), lambda i: (0, 0, 0)),
        ],
        out_specs=[
            pl.BlockSpec((NODE_BLK, D), lambda i: (i, 0)),
            pl.BlockSpec((NODE_BLK, D), lambda i: (i, 0)),
            pl.BlockSpec((NODE_BLK, D), lambda i: (i, 0)),
            pl.BlockSpec((R, NODE_BLK, D), lambda i: (0, i, 0)),
        ],
        out_shape=[
            jax.ShapeDtypeStruct((N, D), _f32),
            jax.ShapeDtypeStruct((N, D), _f32),
            jax.ShapeDtypeStruct((N, D), _f32),
            jax.ShapeDtypeStruct((R, N, D), _f32),
        ],
    )(w_comp.reshape(1, R * NB), h, A_w, A_b.reshape(1, D), B_w,
      B_b.reshape(1, D), E_w, E_b.reshape(1, D), weight)
    return out


def _gather_body(hall_ref, eh_ref, bh_ref, src_ref, dst_ref, ety_ref,
                 msg_o, ehd_o, bhs_o, i_src, i_ety, i_dst, i_msg, rows, sem):
    wid = lax.axis_index("s") * 2 + lax.axis_index("c")
    base = wid * EPW

    def chunk(j, carry):
        off = base + jnp.minimum(j * CHUNK, EPW - CHUNK)
        pltpu.sync_copy(src_ref.at[pl.ds(off, CHUNK)], i_src)
        pltpu.sync_copy(ety_ref.at[pl.ds(off, CHUNK)], i_ety)
        pltpu.sync_copy(dst_ref.at[pl.ds(off, CHUNK)], i_dst)

        def flat(k, c2):
            sl = pl.ds(k * 16, 16)
            i_msg[sl] = i_ety[sl] * N + i_src[sl]
            return c2
        lax.fori_loop(0, CHUNK // 16, flat, 0)

        pltpu.async_copy(hall_ref.at[i_msg], rows, sem).wait()
        pltpu.sync_copy(rows, msg_o.at[pl.ds(off, CHUNK)])
        pltpu.async_copy(eh_ref.at[i_dst], rows, sem).wait()
        pltpu.sync_copy(rows, ehd_o.at[pl.ds(off, CHUNK)])
        pltpu.async_copy(bh_ref.at[i_src], rows, sem).wait()
        pltpu.sync_copy(rows, bhs_o.at[pl.ds(off, CHUNK)])
        return carry

    nch = (EPW + CHUNK - 1) // CHUNK
    lax.fori_loop(0, nch, chunk, 0)


def _sc_gather(hall_flat, eh, bh, src, dst, ety):
    mesh = plsc.VectorSubcoreMesh(core_axis_name="c", subcore_axis_name="s")
    fn = pl.kernel(
        _gather_body,
        out_type=[
            jax.ShapeDtypeStruct((E, D), _f32),
            jax.ShapeDtypeStruct((E, D), _f32),
            jax.ShapeDtypeStruct((E, D), _f32),
        ],
        mesh=mesh,
        scratch_types=[
            pltpu.VMEM((CHUNK,), jnp.int32),
            pltpu.VMEM((CHUNK,), jnp.int32),
            pltpu.VMEM((CHUNK,), jnp.int32),
            pltpu.VMEM((CHUNK,), jnp.int32),
            pltpu.VMEM((CHUNK, D), _f32),
            pltpu.SemaphoreType.DMA,
        ],
    )
    return fn(hall_flat, eh, bh, src, dst, ety)


def _edge_body(e_ref, msg_ref, ehd_ref, bhs_ref, cw_ref, cb_ref,
               g_ref, b_ref, m_ref, v_ref, enew_o, numc_o, sig_o):
    eb = e_ref[...]
    ce = jnp.dot(eb, cw_ref[...], preferred_element_type=_f32) + cb_ref[...]
    eij = ce + msg_ref[...] + ehd_ref[...]
    sig = jax.nn.sigmoid(eij)
    numc = sig * bhs_ref[...]
    numc_o[0] = numc[:, :HALF]
    numc_o[1] = numc[:, HALF:]
    sig_o[0] = sig[:, :HALF]
    sig_o[1] = sig[:, HALF:]
    x = eij * _f32(1.0 / math.sqrt(E))
    y = (x - m_ref[...]) * lax.rsqrt(v_ref[...] + BN_EPS) * g_ref[...] + b_ref[...]
    enew_o[...] = eb + jnp.maximum(y, 0.0)


def _edge_fused(e, msg, ehd, bhs, C_w, C_b, g, b, m, v):
    eblk = E // EDGE_BLK
    blk = pl.BlockSpec((EDGE_BLK, D), lambda i: (i, 0))
    vec = pl.BlockSpec((1, D), lambda i: (0, 0))
    half = pl.BlockSpec((2, EDGE_BLK, HALF), lambda i: (0, i, 0))
    return pl.pallas_call(
        _edge_body,
        grid=(eblk,),
        in_specs=[blk, blk, blk, blk,
                  pl.BlockSpec((D, D), lambda i: (0, 0)),
                  vec, vec, vec, vec, vec],
        out_specs=[blk, half, half],
        out_shape=[
            jax.ShapeDtypeStruct((E, D), _f32),
            jax.ShapeDtypeStruct((2, E, HALF), _f32),
            jax.ShapeDtypeStruct((2, E, HALF), _f32),
        ],
    )(e, msg, ehd, bhs, C_w, C_b.reshape(1, D), g.reshape(1, D),
      b.reshape(1, D), m.reshape(1, D), v.reshape(1, D))


def _scatter_body(numc_ref, sig_ref, dst_ref, out_ref,
                  acc, vals, vals_t, idx, idx_t, zbuf):
    c = lax.axis_index("c")
    s = lax.axis_index("s")

    def zrow(k, carry):
        i = k // 8
        j = (k % 8) * 16
        zbuf[i, pl.ds(j, 16)] = jnp.zeros((16,), _f32)
        return carry
    lax.fori_loop(0, (RPT // 5) * 8, zrow, 0)

    ebase = s * EPT
    rbase = s * RPT
    for q, inref in ((0, numc_ref), (1, sig_ref)):
        def zero(k, carry):
            pltpu.sync_copy(zbuf, acc.at[pl.ds(rbase + k * (RPT // 5), RPT // 5)])
            return carry
        lax.fori_loop(0, 5, zero, 0)
        plsc.subcore_barrier()

        def chunk(j, carry):
            off = ebase + j * CHUNK
            pltpu.sync_copy(dst_ref.at[pl.ds(off, CHUNK)], idx)
            pltpu.sync_copy(inref.at[c, pl.ds(off, CHUNK), :], vals)
            pltpu.sync_copy(vals, acc.at[idx], add=True)
            return carry
        lax.fori_loop(0, EPT // CHUNK, chunk, 0)
        toff = ebase + (EPT // CHUNK) * CHUNK
        pltpu.sync_copy(dst_ref.at[pl.ds(toff, 16)], idx_t)
        pltpu.sync_copy(inref.at[c, pl.ds(toff, 16), :], vals_t)
        pltpu.sync_copy(vals_t, acc.at[idx_t], add=True)
        plsc.subcore_barrier()

        pltpu.sync_copy(acc.at[pl.ds(rbase, RPT)],
                        out_ref.at[q, c, pl.ds(rbase, RPT), :])


def _sc_scatter(numc, sig, dst):
    mesh = plsc.VectorSubcoreMesh(core_axis_name="c", subcore_axis_name="s")
    fn = pl.kernel(
        _scatter_body,
        out_type=jax.ShapeDtypeStruct((2, 2, N, HALF), _f32),
        mesh=mesh,
        scratch_types=[
            pltpu.VMEM_SHARED((N, HALF), _f32),
            pltpu.VMEM((CHUNK, HALF), _f32),
            pltpu.VMEM((16, HALF), _f32),
            pltpu.VMEM((CHUNK,), jnp.int32),
            pltpu.VMEM((16,), jnp.int32),
            pltpu.VMEM((RPT // 5, HALF), _f32),
        ],
    )
    return fn(numc, sig, dst)


def _fin_body(h_ref, ah_ref, sums_ref, g_ref, b_ref, m_ref, v_ref, out_ref):
    sm = sums_ref[...]
    num = jnp.concatenate([sm[0, 0], sm[0, 1]], axis=1)
    den = jnp.concatenate([sm[1, 0], sm[1, 1]], axis=1)
    hb = h_ref[...]
    hagg = ah_ref[...] + num / (den + 1e-6)
    hnew = jnp.where(den > 0.0, hagg, hb)
    x = hnew * _f32(1.0 / math.sqrt(N))
    y = (x - m_ref[...]) * lax.rsqrt(v_ref[...] + BN_EPS) * g_ref[...] + b_ref[...]
    out_ref[...] = hb + jnp.maximum(y, 0.0)


def _node_finalize(h, ah, sums, g, b, m, v):
    nblk = N // NODE_BLK
    blk = pl.BlockSpec((NODE_BLK, D), lambda i: (i, 0))
    vec = pl.BlockSpec((1, D), lambda i: (0, 0))
    return pl.pallas_call(
        _fin_body,
        grid=(nblk,),
        in_specs=[blk, blk,
                  pl.BlockSpec((2, 2, NODE_BLK, HALF), lambda i: (0, 0, i, 0)),
                  vec, vec, vec, vec],
        out_specs=blk,
        out_shape=jax.ShapeDtypeStruct((N, D), _f32),
    )(h, ah, sums, g.reshape(1, D), b.reshape(1, D), m.reshape(1, D),
      v.reshape(1, D))


def kernel(h, e, edge_index, etype, A_w, A_b, B_w, B_b, C_w, C_b, E_w, E_b,
           weight, w_comp, bn_h_gamma, bn_h_beta, bn_h_mean, bn_h_var,
           bn_e_gamma, bn_e_beta, bn_e_mean, bn_e_var):
    src = edge_index[0]
    dst = edge_index[1]
    ah, bh, eh, hall = _node_proj(h, A_w, A_b, B_w, B_b, E_w, E_b,
                                  weight, w_comp)
    msg, ehd, bhs = _sc_gather(hall.reshape(R * N, D), eh, bh, src, dst, etype)
    e_new, numc, sig = _edge_fused(e, msg, ehd, bhs, C_w, C_b,
                                   bn_e_gamma, bn_e_beta, bn_e_mean, bn_e_var)
    sums = _sc_scatter(numc, sig, dst)
    h_new = _node_finalize(h, ah, sums, bn_h_gamma, bn_h_beta,
                           bn_h_mean, bn_h_var)
    return (h_new, e_new)


# speedup vs baseline: 2.0659x; 2.0659x over previous
import math

import jax
import jax.numpy as jnp
from jax import lax
from jax.experimental import pallas as pl
from jax.experimental.pallas import tpu as pltpu
from jax.experimental.pallas import tpu_sc as plsc

N = 10000
E = 160000
D = 256
HALF = 128
R = 8
NB = 4
BN_EPS = 1e-5

NODE_BLK = 400
EDGE_BLK = 640
NW = 32
EPW = E // NW
CHUNK = 128
EPT = E // 16
RPT = 624
ZROWS = 208

_f32 = jnp.float32


def _node_proj_body(wc_ref, h_ref, aw_ref, ab_ref, bw_ref, bb_ref, ew_ref,
                    eb_ref, wt_ref, ah_o, bh_o, eh_o, hall_o):
    hb = h_ref[...]
    ah_o[...] = jnp.dot(hb, aw_ref[...], preferred_element_type=_f32) + ab_ref[...]
    bh_o[...] = jnp.dot(hb, bw_ref[...], preferred_element_type=_f32) + bb_ref[...]
    eh_o[...] = jnp.dot(hb, ew_ref[...], preferred_element_type=_f32) + eb_ref[...]
    base = [jnp.dot(hb, wt_ref[b], preferred_element_type=_f32) for b in range(NB)]
    for r in range(R):
        acc = base[0] * wc_ref[0, r * NB]
        for b in range(1, NB):
            acc = acc + base[b] * wc_ref[0, r * NB + b]
        hall_o[r] = acc


def _node_proj(h, A_w, A_b, B_w, B_b, E_w, E_b, weight, w_comp):
    nblk = N // NODE_BLK
    vec_spec = pl.BlockSpec((1, D), lambda i: (0, 0))
    mat_spec = pl.BlockSpec((D, D), lambda i: (0, 0))
    out = pl.pallas_call(
        _node_proj_body,
        grid=(nblk,),
        in_specs=[
            pl.BlockSpec(memory_space=pltpu.SMEM),
            pl.BlockSpec((NODE_BLK, D), lambda i: (i, 0)),
            mat_spec, vec_spec, mat_spec, vec_spec, mat_spec, vec_spec,
            pl.BlockSpec((NB, D, D), lambda i: (0, 0, 0)),
        ],
        out_specs=[
            pl.BlockSpec((NODE_BLK, D), lambda i: (i, 0)),
            pl.BlockSpec((NODE_BLK, D), lambda i: (i, 0)),
            pl.BlockSpec((NODE_BLK, D), lambda i: (i, 0)),
            pl.BlockSpec((R, NODE_BLK, D), lambda i: (0, i, 0)),
        ],
        out_shape=[
            jax.ShapeDtypeStruct((N, D), _f32),
            jax.ShapeDtypeStruct((N, D), _f32),
            jax.ShapeDtypeStruct((N, D), _f32),
            jax.ShapeDtypeStruct((R, N, D), _f32),
        ],
    )(w_comp.reshape(1, R * NB), h, A_w, A_b.reshape(1, D), B_w,
      B_b.reshape(1, D), E_w, E_b.reshape(1, D), weight)
    return out


def _gather_body(hall_ref, eh_ref, bh_ref, src_ref, dst_ref, ety_ref,
                 msg_o, ehd_o, bhs_o, i_src, i_ety, i_dst, i_msg, rows, sem):
    wid = lax.axis_index("s") * 2 + lax.axis_index("c")
    base = wid * EPW

    def chunk(j, carry):
        off = base + jnp.minimum(j * CHUNK, EPW - CHUNK)
        pltpu.sync_copy(src_ref.at[pl.ds(off, CHUNK)], i_src)
        pltpu.sync_copy(ety_ref.at[pl.ds(off, CHUNK)], i_ety)
        pltpu.sync_copy(dst_ref.at[pl.ds(off, CHUNK)], i_dst)

        def flat(k, c2):
            sl = pl.ds(k * 16, 16)
            i_msg[sl] = i_ety[sl] * N + i_src[sl]
            return c2
        lax.fori_loop(0, CHUNK // 16, flat, 0)

        pltpu.async_copy(hall_ref.at[i_msg], rows, sem).wait()
        pltpu.sync_copy(rows, msg_o.at[pl.ds(off, CHUNK)])
        pltpu.async_copy(eh_ref.at[i_dst], rows, sem).wait()
        pltpu.sync_copy(rows, ehd_o.at[pl.ds(off, CHUNK)])
        pltpu.async_copy(bh_ref.at[i_src], rows, sem).wait()
        pltpu.sync_copy(rows, bhs_o.at[pl.ds(off, CHUNK)])
        return carry

    nch = (EPW + CHUNK - 1) // CHUNK
    lax.fori_loop(0, nch, chunk, 0)


def _sc_gather(hall_flat, eh, bh, src, dst, ety):
    mesh = plsc.VectorSubcoreMesh(core_axis_name="c", subcore_axis_name="s")
    fn = pl.kernel(
        _gather_body,
        out_type=[
            jax.ShapeDtypeStruct((E, D), _f32),
            jax.ShapeDtypeStruct((E, D), _f32),
            jax.ShapeDtypeStruct((E, D), _f32),
        ],
        mesh=mesh,
        scratch_types=[
            pltpu.VMEM((CHUNK,), jnp.int32),
            pltpu.VMEM((CHUNK,), jnp.int32),
            pltpu.VMEM((CHUNK,), jnp.int32),
            pltpu.VMEM((CHUNK,), jnp.int32),
            pltpu.VMEM((CHUNK, D), _f32),
            pltpu.SemaphoreType.DMA,
        ],
    )
    return fn(hall_flat, eh, bh, src, dst, ety)


def _edge_body(e_ref, msg_ref, ehd_ref, bhs_ref, cw_ref, cb_ref,
               g_ref, b_ref, m_ref, v_ref, enew_o, numc_o, sig_o):
    eb = e_ref[...]
    ce = jnp.dot(eb, cw_ref[...], preferred_element_type=_f32) + cb_ref[...]
    eij = ce + msg_ref[...] + ehd_ref[...]
    sig = jax.nn.sigmoid(eij)
    numc = sig * bhs_ref[...]
    numc_o[0] = numc[:, :HALF]
    numc_o[1] = numc[:, HALF:]
    sig_o[0] = sig[:, :HALF]
    sig_o[1] = sig[:, HALF:]
    x = eij * _f32(1.0 / math.sqrt(E))
    y = (x - m_ref[...]) * lax.rsqrt(v_ref[...] + BN_EPS) * g_ref[...] + b_ref[...]
    enew_o[...] = eb + jnp.maximum(y, 0.0)


def _edge_fused(e, msg, ehd, bhs, C_w, C_b, g, b, m, v):
    eblk = E // EDGE_BLK
    blk = pl.BlockSpec((EDGE_BLK, D), lambda i: (i, 0))
    vec = pl.BlockSpec((1, D), lambda i: (0, 0))
    half = pl.BlockSpec((2, EDGE_BLK, HALF), lambda i: (0, i, 0))
    return pl.pallas_call(
        _edge_body,
        grid=(eblk,),
        in_specs=[blk, blk, blk, blk,
                  pl.BlockSpec((D, D), lambda i: (0, 0)),
                  vec, vec, vec, vec, vec],
        out_specs=[blk, half, half],
        out_shape=[
            jax.ShapeDtypeStruct((E, D), _f32),
            jax.ShapeDtypeStruct((2, E, HALF), _f32),
            jax.ShapeDtypeStruct((2, E, HALF), _f32),
        ],
    )(e, msg, ehd, bhs, C_w, C_b.reshape(1, D), g.reshape(1, D),
      b.reshape(1, D), m.reshape(1, D), v.reshape(1, D))


def _scatter_body(numc_ref, sig_ref, dst_ref, out_ref,
                  acc, vals, vals_t, idx, idx_t, zbuf):
    c = lax.axis_index("c")
    s = lax.axis_index("s")
    last = s == 15

    def zrow(k, carry):
        i = k // 8
        j = (k % 8) * 16
        zbuf[i, pl.ds(j, 16)] = jnp.zeros((16,), _f32)
        return carry
    lax.fori_loop(0, ZROWS * 8, zrow, 0)

    ebase = s * EPT
    rbase = s * RPT
    for q, inref in ((0, numc_ref), (1, sig_ref)):
        def zero(k, carry):
            pltpu.sync_copy(zbuf, acc.at[pl.ds(rbase + k * ZROWS, ZROWS)])
            return carry
        lax.fori_loop(0, RPT // ZROWS, zero, 0)
        pl.when(last)(lambda: pltpu.sync_copy(
            zbuf.at[pl.ds(0, 16)], acc.at[pl.ds(16 * RPT, 16)]))
        plsc.subcore_barrier()

        def chunk(j, carry):
            off = ebase + j * CHUNK
            pltpu.sync_copy(dst_ref.at[pl.ds(off, CHUNK)], idx)
            pltpu.sync_copy(inref.at[c, pl.ds(off, CHUNK), :], vals)
            pltpu.sync_copy(vals, acc.at[idx], add=True)
            return carry
        lax.fori_loop(0, EPT // CHUNK, chunk, 0)
        toff = ebase + (EPT // CHUNK) * CHUNK
        pltpu.sync_copy(dst_ref.at[pl.ds(toff, 16)], idx_t)
        pltpu.sync_copy(inref.at[c, pl.ds(toff, 16), :], vals_t)
        pltpu.sync_copy(vals_t, acc.at[idx_t], add=True)
        plsc.subcore_barrier()

        pltpu.sync_copy(acc.at[pl.ds(rbase, RPT)],
                        out_ref.at[q, c, pl.ds(rbase, RPT), :])
        pl.when(last)(lambda: pltpu.sync_copy(
            acc.at[pl.ds(16 * RPT, 16)],
            out_ref.at[q, c, pl.ds(16 * RPT, 16), :]))


def _sc_scatter(numc, sig, dst):
    mesh = plsc.VectorSubcoreMesh(core_axis_name="c", subcore_axis_name="s")
    fn = pl.kernel(
        _scatter_body,
        out_type=jax.ShapeDtypeStruct((2, 2, N, HALF), _f32),
        mesh=mesh,
        scratch_types=[
            pltpu.VMEM_SHARED((N, HALF), _f32),
            pltpu.VMEM((CHUNK, HALF), _f32),
            pltpu.VMEM((16, HALF), _f32),
            pltpu.VMEM((CHUNK,), jnp.int32),
            pltpu.VMEM((16,), jnp.int32),
            pltpu.VMEM((ZROWS, HALF), _f32),
        ],
    )
    return fn(numc, sig, dst)


def _fin_body(h_ref, ah_ref, sums_ref, g_ref, b_ref, m_ref, v_ref, out_ref):
    sm = sums_ref[...]
    num = jnp.concatenate([sm[0, 0], sm[0, 1]], axis=1)
    den = jnp.concatenate([sm[1, 0], sm[1, 1]], axis=1)
    hb = h_ref[...]
    hagg = ah_ref[...] + num / (den + 1e-6)
    hnew = jnp.where(den > 0.0, hagg, hb)
    x = hnew * _f32(1.0 / math.sqrt(N))
    y = (x - m_ref[...]) * lax.rsqrt(v_ref[...] + BN_EPS) * g_ref[...] + b_ref[...]
    out_ref[...] = hb + jnp.maximum(y, 0.0)


def _node_finalize(h, ah, sums, g, b, m, v):
    nblk = N // NODE_BLK
    blk = pl.BlockSpec((NODE_BLK, D), lambda i: (i, 0))
    vec = pl.BlockSpec((1, D), lambda i: (0, 0))
    return pl.pallas_call(
        _fin_body,
        grid=(nblk,),
        in_specs=[blk, blk,
                  pl.BlockSpec((2, 2, NODE_BLK, HALF), lambda i: (0, 0, i, 0)),
                  vec, vec, vec, vec],
        out_specs=blk,
        out_shape=jax.ShapeDtypeStruct((N, D), _f32),
    )(h, ah, sums, g.reshape(1, D), b.reshape(1, D), m.reshape(1, D),
      v.reshape(1, D))


def kernel(h, e, edge_index, etype, A_w, A_b, B_w, B_b, C_w, C_b, E_w, E_b,
           weight, w_comp, bn_h_gamma, bn_h_beta, bn_h_mean, bn_h_var,
           bn_e_gamma, bn_e_beta, bn_e_mean, bn_e_var):
    src = edge_index[0]
    dst = edge_index[1]
    ah, bh, eh, hall = _node_proj(h, A_w, A_b, B_w, B_b, E_w, E_b,
                                  weight, w_comp)
    msg, ehd, bhs = _sc_gather(hall.reshape(R * N, D), eh, bh, src, dst, etype)
    e_new, numc, sig = _edge_fused(e, msg, ehd, bhs, C_w, C_b,
                                   bn_e_gamma, bn_e_beta, bn_e_mean, bn_e_var)
    sums = _sc_scatter(numc, sig, dst)
    h_new = _node_finalize(h, ah, sums, bn_h_gamma, bn_h_beta,
                           bn_h_mean, bn_h_var)
    return (h_new, e_new)


# speedup vs baseline: 2.7497x; 1.3310x over previous
import math

import jax
import jax.numpy as jnp
from jax import lax
from jax.experimental import pallas as pl
from jax.experimental.pallas import tpu as pltpu
from jax.experimental.pallas import tpu_sc as plsc

N = 10000
E = 160000
D = 256
HALF = 128
R = 8
NB = 4
BN_EPS = 1e-5

NODE_BLK = 400
EDGE_BLK = 640
NW = 32
EPW = E // NW
CHUNK = 128
EPT = E // 16
RPT = 624
ZROWS = 104

_f32 = jnp.float32


def _node_proj_body(wc_ref, h_ref, aw_ref, ab_ref, bw_ref, bb_ref, ew_ref,
                    eb_ref, wt_ref, ah_o, bh_o, eh_o, hall_o):
    hb = h_ref[...]
    ah_o[...] = jnp.dot(hb, aw_ref[...], preferred_element_type=_f32) + ab_ref[...]
    bh_o[...] = jnp.dot(hb, bw_ref[...], preferred_element_type=_f32) + bb_ref[...]
    eh_o[...] = jnp.dot(hb, ew_ref[...], preferred_element_type=_f32) + eb_ref[...]
    base = [jnp.dot(hb, wt_ref[b], preferred_element_type=_f32) for b in range(NB)]
    for r in range(R):
        acc = base[0] * wc_ref[0, r * NB]
        for b in range(1, NB):
            acc = acc + base[b] * wc_ref[0, r * NB + b]
        hall_o[r] = acc


def _node_proj(h, A_w, A_b, B_w, B_b, E_w, E_b, weight, w_comp):
    nblk = N // NODE_BLK
    vec_spec = pl.BlockSpec((1, D), lambda i: (0, 0))
    mat_spec = pl.BlockSpec((D, D), lambda i: (0, 0))
    out = pl.pallas_call(
        _node_proj_body,
        grid=(nblk,),
        in_specs=[
            pl.BlockSpec(memory_space=pltpu.SMEM),
            pl.BlockSpec((NODE_BLK, D), lambda i: (i, 0)),
            mat_spec, vec_spec, mat_spec, vec_spec, mat_spec, vec_spec,
            pl.BlockSpec((NB, D, D), lambda i: (0, 0, 0)),
        ],
        out_specs=[
            pl.BlockSpec((NODE_BLK, D), lambda i: (i, 0)),
            pl.BlockSpec((NODE_BLK, D), lambda i: (i, 0)),
            pl.BlockSpec((NODE_BLK, D), lambda i: (i, 0)),
            pl.BlockSpec((R, NODE_BLK, D), lambda i: (0, i, 0)),
        ],
        out_shape=[
            jax.ShapeDtypeStruct((N, D), _f32),
            jax.ShapeDtypeStruct((N, D), _f32),
            jax.ShapeDtypeStruct((N, D), _f32),
            jax.ShapeDtypeStruct((R, N, D), _f32),
        ],
    )(w_comp.reshape(1, R * NB), h, A_w, A_b.reshape(1, D), B_w,
      B_b.reshape(1, D), E_w, E_b.reshape(1, D), weight)
    return out


GCH = 48
GCHUNKS = 105
GGROUPS = GCHUNKS // 3


def _gather_body(hall_ref, eh_ref, bh_ref, src_ref, dst_ref, ety_ref,
                 msg_o, ehd_o, bhs_o, *scr):
    bufs = [scr[b * 7:(b + 1) * 7] for b in range(3)]
    sems = [scr[21 + b * 7: 21 + (b + 1) * 7] for b in range(3)]
    wid = lax.axis_index("s") * 2 + lax.axis_index("c")
    base = wid * EPW

    def off_of(j):
        return base + jnp.minimum(j * GCH, EPW - GCH)

    def issue_idx(j, b):
        i_src, i_ety, i_dst = bufs[b][0], bufs[b][1], bufs[b][2]
        semi = sems[b][0]
        off = off_of(j)
        pltpu.async_copy(src_ref.at[pl.ds(off, GCH)], i_src, semi)
        pltpu.async_copy(ety_ref.at[pl.ds(off, GCH)], i_ety, semi)
        pltpu.async_copy(dst_ref.at[pl.ds(off, GCH)], i_dst, semi)

    def wait_idx(j, b):
        i_src, i_ety, i_dst = bufs[b][0], bufs[b][1], bufs[b][2]
        semi = sems[b][0]
        off = off_of(j)
        pltpu.make_async_copy(src_ref.at[pl.ds(off, GCH)], i_src, semi).wait()
        pltpu.make_async_copy(ety_ref.at[pl.ds(off, GCH)], i_ety, semi).wait()
        pltpu.make_async_copy(dst_ref.at[pl.ds(off, GCH)], i_dst, semi).wait()

    def step(j, b, wait_wb):
        i_src, i_ety, i_dst, i_msg, rmsg, rehd, rbhs = bufs[b]
        semi, semg1, semg2, semg3, semw1, semw2, semw3 = sems[b]
        wait_idx(j, b)
        for k in range(GCH // 16):
            sl = pl.ds(k * 16, 16)
            i_msg[sl] = i_ety[sl] * N + i_src[sl]
        if wait_wb:
            po = pl.ds(off_of(j - 3), GCH)
            pltpu.make_async_copy(rmsg, msg_o.at[po], semw1).wait()
            pltpu.make_async_copy(rehd, ehd_o.at[po], semw2).wait()
            pltpu.make_async_copy(rbhs, bhs_o.at[po], semw3).wait()
        off = off_of(j)
        d1 = pltpu.async_copy(hall_ref.at[i_msg], rmsg, semg1)
        d2 = pltpu.async_copy(eh_ref.at[i_dst], rehd, semg2)
        d3 = pltpu.async_copy(bh_ref.at[i_src], rbhs, semg3)
        d1.wait()
        pltpu.async_copy(rmsg, msg_o.at[pl.ds(off, GCH)], semw1)
        d2.wait()
        pltpu.async_copy(rehd, ehd_o.at[pl.ds(off, GCH)], semw2)
        d3.wait()
        pltpu.async_copy(rbhs, bhs_o.at[pl.ds(off, GCH)], semw3)
        issue_idx(j + 3, b)

    for b in range(3):
        issue_idx(b, b)
    for b in range(3):
        step(b, b, False)

    def group(g, carry):
        for b in range(3):
            step(g * 3 + b, b, True)
        return carry
    lax.fori_loop(1, GGROUPS, group, 0)

    for b in range(3):
        j = GCHUNKS - 3 + b
        po = pl.ds(off_of(j), GCH)
        pltpu.make_async_copy(bufs[b][4], msg_o.at[po], sems[b][4]).wait()
        pltpu.make_async_copy(bufs[b][5], ehd_o.at[po], sems[b][5]).wait()
        pltpu.make_async_copy(bufs[b][6], bhs_o.at[po], sems[b][6]).wait()
        wait_idx(j + 3, b)


def _sc_gather(hall_flat, eh, bh, src, dst, ety):
    mesh = plsc.VectorSubcoreMesh(core_axis_name="c", subcore_axis_name="s")
    scratch = []
    for _ in range(3):
        scratch += [pltpu.VMEM((GCH,), jnp.int32)] * 4
        scratch += [pltpu.VMEM((GCH, D), _f32)] * 3
    scratch += [pltpu.SemaphoreType.DMA] * 21
    fn = pl.kernel(
        _gather_body,
        out_type=[
            jax.ShapeDtypeStruct((E, D), _f32),
            jax.ShapeDtypeStruct((E, D), _f32),
            jax.ShapeDtypeStruct((E, D), _f32),
        ],
        mesh=mesh,
        scratch_types=scratch,
    )
    return fn(hall_flat, eh, bh, src, dst, ety)


def _edge_body(e_ref, msg_ref, ehd_ref, bhs_ref, cw_ref, cb_ref,
               g_ref, b_ref, m_ref, v_ref, enew_o, numc_o, sig_o):
    eb = e_ref[...]
    ce = jnp.dot(eb, cw_ref[...], preferred_element_type=_f32) + cb_ref[...]
    eij = ce + msg_ref[...] + ehd_ref[...]
    sig = jax.nn.sigmoid(eij)
    numc = sig * bhs_ref[...]
    numc_o[0] = numc[:, :HALF]
    numc_o[1] = numc[:, HALF:]
    sig_o[0] = sig[:, :HALF]
    sig_o[1] = sig[:, HALF:]
    x = eij * _f32(1.0 / math.sqrt(E))
    y = (x - m_ref[...]) * lax.rsqrt(v_ref[...] + BN_EPS) * g_ref[...] + b_ref[...]
    enew_o[...] = eb + jnp.maximum(y, 0.0)


def _edge_fused(e, msg, ehd, bhs, C_w, C_b, g, b, m, v):
    eblk = E // EDGE_BLK
    blk = pl.BlockSpec((EDGE_BLK, D), lambda i: (i, 0))
    vec = pl.BlockSpec((1, D), lambda i: (0, 0))
    half = pl.BlockSpec((2, EDGE_BLK, HALF), lambda i: (0, i, 0))
    return pl.pallas_call(
        _edge_body,
        grid=(eblk,),
        in_specs=[blk, blk, blk, blk,
                  pl.BlockSpec((D, D), lambda i: (0, 0)),
                  vec, vec, vec, vec, vec],
        out_specs=[blk, half, half],
        out_shape=[
            jax.ShapeDtypeStruct((E, D), _f32),
            jax.ShapeDtypeStruct((2, E, HALF), _f32),
            jax.ShapeDtypeStruct((2, E, HALF), _f32),
        ],
    )(e, msg, ehd, bhs, C_w, C_b.reshape(1, D), g.reshape(1, D),
      b.reshape(1, D), m.reshape(1, D), v.reshape(1, D))


SCHUNKS = EPT // CHUNK
SRING = 2
SGROUPS = SCHUNKS // SRING


def _scatter_body(numc_ref, sig_ref, dst_ref, out_ref,
                  acc, zbuf, idx_t, vals_t, *scr):
    idxs = [scr[b] for b in range(SRING)]
    vals = [scr[SRING + b] for b in range(SRING)]
    semin = [scr[2 * SRING + b] for b in range(SRING)]
    semadd = [scr[3 * SRING + b] for b in range(SRING)]
    c = lax.axis_index("c")
    s = lax.axis_index("s")
    last = s == 15

    def zrow(k, carry):
        i = k // 8
        j = (k % 8) * 16
        zbuf[i, pl.ds(j, 16)] = jnp.zeros((16,), _f32)
        return carry
    lax.fori_loop(0, ZROWS * 8, zrow, 0)

    ebase = s * EPT
    rbase = s * RPT
    for q, inref in ((0, numc_ref), (1, sig_ref)):
        def zero(k, carry):
            pltpu.sync_copy(zbuf, acc.at[pl.ds(rbase + k * ZROWS, ZROWS)])
            return carry
        lax.fori_loop(0, RPT // ZROWS, zero, 0)
        pl.when(last)(lambda: pltpu.sync_copy(
            zbuf.at[pl.ds(0, 16)], acc.at[pl.ds(16 * RPT, 16)]))
        plsc.subcore_barrier()

        def off_of(j):
            return ebase + jnp.minimum(j * CHUNK, (SCHUNKS - 1) * CHUNK)

        def issue_loads(j, b):
            off = off_of(j)
            pltpu.async_copy(dst_ref.at[pl.ds(off, CHUNK)], idxs[b], semin[b])
            pltpu.async_copy(inref.at[c, pl.ds(off, CHUNK), :], vals[b],
                             semin[b])

        def wait_loads(j, b):
            off = off_of(j)
            pltpu.make_async_copy(
                dst_ref.at[pl.ds(off, CHUNK)], idxs[b], semin[b]).wait()
            pltpu.make_async_copy(
                inref.at[c, pl.ds(off, CHUNK), :], vals[b], semin[b]).wait()

        for b in range(SRING):
            issue_loads(b, b)

        def group(g, carry):
            for b in range(SRING):
                j = g * SRING + b
                wait_loads(j, b)
                pltpu.async_copy(vals[b], acc.at[idxs[b]], semadd[b],
                                 add=True).wait()
                issue_loads(j + SRING, b)
            return carry
        lax.fori_loop(0, SGROUPS, group, 0)
        for b in range(SRING):
            wait_loads(SCHUNKS + b, b)

        toff = ebase + SCHUNKS * CHUNK
        pltpu.sync_copy(dst_ref.at[pl.ds(toff, 16)], idx_t)
        pltpu.sync_copy(inref.at[c, pl.ds(toff, 16), :], vals_t)
        pltpu.sync_copy(vals_t, acc.at[idx_t], add=True)
        plsc.subcore_barrier()

        pltpu.sync_copy(acc.at[pl.ds(rbase, RPT)],
                        out_ref.at[q, c, pl.ds(rbase, RPT), :])
        pl.when(last)(lambda: pltpu.sync_copy(
            acc.at[pl.ds(16 * RPT, 16)],
            out_ref.at[q, c, pl.ds(16 * RPT, 16), :]))


def _sc_scatter(numc, sig, dst):
    mesh = plsc.VectorSubcoreMesh(core_axis_name="c", subcore_axis_name="s")
    scratch = [
        pltpu.VMEM_SHARED((N, HALF), _f32),
        pltpu.VMEM((ZROWS, HALF), _f32),
        pltpu.VMEM((16,), jnp.int32),
        pltpu.VMEM((16, HALF), _f32),
    ]
    scratch += [pltpu.VMEM((CHUNK,), jnp.int32)] * SRING
    scratch += [pltpu.VMEM((CHUNK, HALF), _f32)] * SRING
    scratch += [pltpu.SemaphoreType.DMA] * (2 * SRING)
    fn = pl.kernel(
        _scatter_body,
        out_type=jax.ShapeDtypeStruct((2, 2, N, HALF), _f32),
        mesh=mesh,
        scratch_types=scratch,
    )
    return fn(numc, sig, dst)


def _fin_body(h_ref, ah_ref, sums_ref, g_ref, b_ref, m_ref, v_ref, out_ref):
    sm = sums_ref[...]
    num = jnp.concatenate([sm[0, 0], sm[0, 1]], axis=1)
    den = jnp.concatenate([sm[1, 0], sm[1, 1]], axis=1)
    hb = h_ref[...]
    hagg = ah_ref[...] + num / (den + 1e-6)
    hnew = jnp.where(den > 0.0, hagg, hb)
    x = hnew * _f32(1.0 / math.sqrt(N))
    y = (x - m_ref[...]) * lax.rsqrt(v_ref[...] + BN_EPS) * g_ref[...] + b_ref[...]
    out_ref[...] = hb + jnp.maximum(y, 0.0)


def _node_finalize(h, ah, sums, g, b, m, v):
    nblk = N // NODE_BLK
    blk = pl.BlockSpec((NODE_BLK, D), lambda i: (i, 0))
    vec = pl.BlockSpec((1, D), lambda i: (0, 0))
    return pl.pallas_call(
        _fin_body,
        grid=(nblk,),
        in_specs=[blk, blk,
                  pl.BlockSpec((2, 2, NODE_BLK, HALF), lambda i: (0, 0, i, 0)),
                  vec, vec, vec, vec],
        out_specs=blk,
        out_shape=jax.ShapeDtypeStruct((N, D), _f32),
    )(h, ah, sums, g.reshape(1, D), b.reshape(1, D), m.reshape(1, D),
      v.reshape(1, D))


def kernel(h, e, edge_index, etype, A_w, A_b, B_w, B_b, C_w, C_b, E_w, E_b,
           weight, w_comp, bn_h_gamma, bn_h_beta, bn_h_mean, bn_h_var,
           bn_e_gamma, bn_e_beta, bn_e_mean, bn_e_var):
    src = edge_index[0]
    dst = edge_index[1]
    ah, bh, eh, hall = _node_proj(h, A_w, A_b, B_w, B_b, E_w, E_b,
                                  weight, w_comp)
    msg, ehd, bhs = _sc_gather(hall.reshape(R * N, D), eh, bh, src, dst, etype)
    e_new, numc, sig = _edge_fused(e, msg, ehd, bhs, C_w, C_b,
                                   bn_e_gamma, bn_e_beta, bn_e_mean, bn_e_var)
    sums = _sc_scatter(numc, sig, dst)
    h_new = _node_finalize(h, ah, sums, bn_h_gamma, bn_h_beta,
                           bn_h_mean, bn_h_var)
    return (h_new, e_new)


# speedup vs baseline: 2.8495x; 1.0363x over previous
import functools
import math

import jax
import jax.numpy as jnp
from jax import lax
from jax.experimental import pallas as pl
from jax.experimental.pallas import tpu as pltpu
from jax.experimental.pallas import tpu_sc as plsc

N = 10000
E = 160000
D = 256
HALF = 128
R = 8
NB = 4
BN_EPS = 1e-5

NODE_BLK = 400
EDGE_BLK = 640
E2 = E // 2
NW = 32
GEPW = 2504
GEPW_LAST = E2 - (NW - 1) * GEPW
SCH = 104
SEPT = E2 // 16
RPT = 624
ZROWS = 104

_f32 = jnp.float32


def _node_proj_body(wc_ref, h_ref, aw_ref, ab_ref, bw_ref, bb_ref, ew_ref,
                    eb_ref, wt_ref, ah_o, bh_o, eh_o, hall_o):
    hb = h_ref[...]
    ah_o[...] = jnp.dot(hb, aw_ref[...], preferred_element_type=_f32) + ab_ref[...]
    bh_o[...] = jnp.dot(hb, bw_ref[...], preferred_element_type=_f32) + bb_ref[...]
    eh_o[...] = jnp.dot(hb, ew_ref[...], preferred_element_type=_f32) + eb_ref[...]
    base = [jnp.dot(hb, wt_ref[b], preferred_element_type=_f32) for b in range(NB)]
    for r in range(R):
        acc = base[0] * wc_ref[0, r * NB]
        for b in range(1, NB):
            acc = acc + base[b] * wc_ref[0, r * NB + b]
        hall_o[r] = acc


def _node_proj(h, A_w, A_b, B_w, B_b, E_w, E_b, weight, w_comp):
    nblk = N // NODE_BLK
    vec_spec = pl.BlockSpec((1, D), lambda i: (0, 0))
    mat_spec = pl.BlockSpec((D, D), lambda i: (0, 0))
    out = pl.pallas_call(
        _node_proj_body,
        grid=(nblk,),
        in_specs=[
            pl.BlockSpec(memory_space=pltpu.SMEM),
            pl.BlockSpec((NODE_BLK, D), lambda i: (i, 0)),
            mat_spec, vec_spec, mat_spec, vec_spec, mat_spec, vec_spec,
            pl.BlockSpec((NB, D, D), lambda i: (0, 0, 0)),
        ],
        out_specs=[
            pl.BlockSpec((NODE_BLK, D), lambda i: (i, 0)),
            pl.BlockSpec((NODE_BLK, D), lambda i: (i, 0)),
            pl.BlockSpec((NODE_BLK, D), lambda i: (i, 0)),
            pl.BlockSpec((R, NODE_BLK, D), lambda i: (0, i, 0)),
        ],
        out_shape=[
            jax.ShapeDtypeStruct((N, D), _f32),
            jax.ShapeDtypeStruct((N, D), _f32),
            jax.ShapeDtypeStruct((N, D), _f32),
            jax.ShapeDtypeStruct((R, N, D), _f32),
        ],
    )(w_comp.reshape(1, R * NB), h, A_w, A_b.reshape(1, D), B_w,
      B_b.reshape(1, D), E_w, E_b.reshape(1, D), weight)
    return out


GCH = 48
GCHUNKS = 54
GGROUPS = GCHUNKS // 3


def _gather_body(lo, hall_ref, eh_ref, bh_ref, src_ref, dst_ref, ety_ref,
                 msg_o, ehd_o, bhs_o, *scr):
    bufs = [scr[b * 7:(b + 1) * 7] for b in range(3)]
    sems = [scr[21 + b * 7: 21 + (b + 1) * 7] for b in range(3)]
    wid = lax.axis_index("s") * 2 + lax.axis_index("c")
    base = wid * GEPW
    epw_w = jnp.where(wid == NW - 1, GEPW_LAST, GEPW)

    def off_of(j):
        return base + jnp.minimum(j * GCH, epw_w - GCH)

    def issue_idx(j, b):
        i_src, i_ety, i_dst = bufs[b][0], bufs[b][1], bufs[b][2]
        semi = sems[b][0]
        off = lo + off_of(j)
        pltpu.async_copy(src_ref.at[pl.ds(off, GCH)], i_src, semi)
        pltpu.async_copy(ety_ref.at[pl.ds(off, GCH)], i_ety, semi)
        pltpu.async_copy(dst_ref.at[pl.ds(off, GCH)], i_dst, semi)

    def wait_idx(j, b):
        i_src, i_ety, i_dst = bufs[b][0], bufs[b][1], bufs[b][2]
        semi = sems[b][0]
        off = lo + off_of(j)
        pltpu.make_async_copy(src_ref.at[pl.ds(off, GCH)], i_src, semi).wait()
        pltpu.make_async_copy(ety_ref.at[pl.ds(off, GCH)], i_ety, semi).wait()
        pltpu.make_async_copy(dst_ref.at[pl.ds(off, GCH)], i_dst, semi).wait()

    def step(j, b, wait_wb):
        i_src, i_ety, i_dst, i_msg, rmsg, rehd, rbhs = bufs[b]
        semi, semg1, semg2, semg3, semw1, semw2, semw3 = sems[b]
        wait_idx(j, b)
        for k in range(GCH // 16):
            sl = pl.ds(k * 16, 16)
            i_msg[sl] = i_ety[sl] * N + i_src[sl]
        if wait_wb:
            po = pl.ds(off_of(j - 3), GCH)
            pltpu.make_async_copy(rmsg, msg_o.at[po], semw1).wait()
            pltpu.make_async_copy(rehd, ehd_o.at[po], semw2).wait()
            pltpu.make_async_copy(rbhs, bhs_o.at[po], semw3).wait()
        off = off_of(j)
        d1 = pltpu.async_copy(hall_ref.at[i_msg], rmsg, semg1)
        d2 = pltpu.async_copy(eh_ref.at[i_dst], rehd, semg2)
        d3 = pltpu.async_copy(bh_ref.at[i_src], rbhs, semg3)
        d1.wait()
        pltpu.async_copy(rmsg, msg_o.at[pl.ds(off, GCH)], semw1)
        d2.wait()
        pltpu.async_copy(rehd, ehd_o.at[pl.ds(off, GCH)], semw2)
        d3.wait()
        pltpu.async_copy(rbhs, bhs_o.at[pl.ds(off, GCH)], semw3)
        issue_idx(j + 3, b)

    for b in range(3):
        issue_idx(b, b)
    for b in range(3):
        step(b, b, False)

    def group(g, carry):
        for b in range(3):
            step(g * 3 + b, b, True)
        return carry
    lax.fori_loop(1, GGROUPS, group, 0)

    for b in range(3):
        j = GCHUNKS - 3 + b
        po = pl.ds(off_of(j), GCH)
        pltpu.make_async_copy(bufs[b][4], msg_o.at[po], sems[b][4]).wait()
        pltpu.make_async_copy(bufs[b][5], ehd_o.at[po], sems[b][5]).wait()
        pltpu.make_async_copy(bufs[b][6], bhs_o.at[po], sems[b][6]).wait()
        wait_idx(j + 3, b)


def _sc_gather(hall_flat, eh, bh, src, dst, ety, lo):
    mesh = plsc.VectorSubcoreMesh(core_axis_name="c", subcore_axis_name="s")
    scratch = []
    for _ in range(3):
        scratch += [pltpu.VMEM((GCH,), jnp.int32)] * 4
        scratch += [pltpu.VMEM((GCH, D), _f32)] * 3
    scratch += [pltpu.SemaphoreType.DMA] * 21
    fn = pl.kernel(
        functools.partial(_gather_body, lo),
        out_type=[
            jax.ShapeDtypeStruct((E2, D), _f32),
            jax.ShapeDtypeStruct((E2, D), _f32),
            jax.ShapeDtypeStruct((E2, D), _f32),
        ],
        mesh=mesh,
        scratch_types=scratch,
        name=f"gather_lo{lo}",
    )
    return fn(hall_flat, eh, bh, src, dst, ety)


def _edge_body(e_ref, msg_ref, ehd_ref, bhs_ref, cw_ref, cb_ref,
               g_ref, b_ref, m_ref, v_ref, enew_o, numc_o, sig_o):
    eb = e_ref[...]
    ce = jnp.dot(eb, cw_ref[...], preferred_element_type=_f32) + cb_ref[...]
    eij = ce + msg_ref[...] + ehd_ref[...]
    sig = jax.nn.sigmoid(eij)
    numc = sig * bhs_ref[...]
    numc_o[0] = numc[:, :HALF]
    numc_o[1] = numc[:, HALF:]
    sig_o[0] = sig[:, :HALF]
    sig_o[1] = sig[:, HALF:]
    x = eij * _f32(1.0 / math.sqrt(E))
    y = (x - m_ref[...]) * lax.rsqrt(v_ref[...] + BN_EPS) * g_ref[...] + b_ref[...]
    enew_o[...] = eb + jnp.maximum(y, 0.0)


def _edge_body_alias(e_ref, msg_ref, ehd_ref, bhs_ref, cw_ref, cb_ref,
                     g_ref, b_ref, m_ref, v_ref, prev_ref,
                     enew_o, numc_o, sig_o):
    _edge_body(e_ref, msg_ref, ehd_ref, bhs_ref, cw_ref, cb_ref,
               g_ref, b_ref, m_ref, v_ref, enew_o, numc_o, sig_o)


def _edge_fused(e, msg, ehd, bhs, C_w, C_b, g, b, m, v, lo, prev):
    eblk = E2 // EDGE_BLK
    boff = lo // EDGE_BLK
    eblk_spec = pl.BlockSpec((EDGE_BLK, D), lambda i: (i + boff, 0))
    blk = pl.BlockSpec((EDGE_BLK, D), lambda i: (i, 0))
    vec = pl.BlockSpec((1, D), lambda i: (0, 0))
    half = pl.BlockSpec((2, EDGE_BLK, HALF), lambda i: (0, i, 0))
    in_specs = [eblk_spec, blk, blk, blk,
                pl.BlockSpec((D, D), lambda i: (0, 0)),
                vec, vec, vec, vec, vec]
    args = [e, msg, ehd, bhs, C_w, C_b.reshape(1, D), g.reshape(1, D),
            b.reshape(1, D), m.reshape(1, D), v.reshape(1, D)]
    body = _edge_body
    io_alias = {}
    if prev is not None:
        in_specs = in_specs + [pl.BlockSpec(memory_space=pl.ANY)]
        args = args + [prev]
        body = _edge_body_alias
        io_alias = {10: 0}
    return pl.pallas_call(
        body,
        grid=(eblk,),
        in_specs=in_specs,
        out_specs=[eblk_spec, half, half],
        out_shape=[
            jax.ShapeDtypeStruct((E, D), _f32),
            jax.ShapeDtypeStruct((2, E2, HALF), _f32),
            jax.ShapeDtypeStruct((2, E2, HALF), _f32),
        ],
        input_output_aliases=io_alias,
    )(*args)


SCHUNKS = 48
SRING = 2
SGROUPS = SCHUNKS // SRING


def _scatter_body(lo, numc_ref, sig_ref, dst_ref, out_ref,
                  acc, zbuf, idx_t, vals_t, *scr):
    idxs = [scr[b] for b in range(SRING)]
    vals = [scr[SRING + b] for b in range(SRING)]
    semin = [scr[2 * SRING + b] for b in range(SRING)]
    semadd = [scr[3 * SRING + b] for b in range(SRING)]
    c = lax.axis_index("c")
    s = lax.axis_index("s")
    last = s == 15

    def zrow(k, carry):
        i = k // 8
        j = (k % 8) * 16
        zbuf[i, pl.ds(j, 16)] = jnp.zeros((16,), _f32)
        return carry
    lax.fori_loop(0, ZROWS * 8, zrow, 0)

    ebase = s * SEPT
    rbase = s * RPT
    for q, inref in ((0, numc_ref), (1, sig_ref)):
        def zero(k, carry):
            pltpu.sync_copy(zbuf, acc.at[pl.ds(rbase + k * ZROWS, ZROWS)])
            return carry
        lax.fori_loop(0, RPT // ZROWS, zero, 0)
        pl.when(last)(lambda: pltpu.sync_copy(
            zbuf.at[pl.ds(0, 16)], acc.at[pl.ds(16 * RPT, 16)]))
        plsc.subcore_barrier()

        def off_of(j):
            return ebase + jnp.minimum(j * SCH, (SCHUNKS - 1) * SCH)

        def issue_loads(j, b):
            off = off_of(j)
            pltpu.async_copy(dst_ref.at[pl.ds(lo + off, SCH)], idxs[b],
                             semin[b])
            pltpu.async_copy(inref.at[c, pl.ds(off, SCH), :], vals[b],
                             semin[b])

        def wait_loads(j, b):
            off = off_of(j)
            pltpu.make_async_copy(
                dst_ref.at[pl.ds(lo + off, SCH)], idxs[b], semin[b]).wait()
            pltpu.make_async_copy(
                inref.at[c, pl.ds(off, SCH), :], vals[b], semin[b]).wait()

        for b in range(SRING):
            issue_loads(b, b)

        def group(g, carry):
            for b in range(SRING):
                j = g * SRING + b
                wait_loads(j, b)
                pltpu.async_copy(vals[b], acc.at[idxs[b]], semadd[b],
                                 add=True).wait()
                issue_loads(j + SRING, b)
            return carry
        lax.fori_loop(0, SGROUPS, group, 0)
        for b in range(SRING):
            wait_loads(SCHUNKS + b, b)

        toff = ebase + SCHUNKS * SCH
        pltpu.sync_copy(dst_ref.at[pl.ds(lo + toff, 8)], idx_t)
        pltpu.sync_copy(inref.at[c, pl.ds(toff, 8), :], vals_t)
        pltpu.sync_copy(vals_t, acc.at[idx_t], add=True)
        plsc.subcore_barrier()

        pltpu.sync_copy(acc.at[pl.ds(rbase, RPT)],
                        out_ref.at[q, c, pl.ds(rbase, RPT), :])
        pl.when(last)(lambda: pltpu.sync_copy(
            acc.at[pl.ds(16 * RPT, 16)],
            out_ref.at[q, c, pl.ds(16 * RPT, 16), :]))


def _sc_scatter(numc, sig, dst, lo):
    mesh = plsc.VectorSubcoreMesh(core_axis_name="c", subcore_axis_name="s")
    scratch = [
        pltpu.VMEM_SHARED((N, HALF), _f32),
        pltpu.VMEM((ZROWS, HALF), _f32),
        pltpu.VMEM((8,), jnp.int32),
        pltpu.VMEM((8, HALF), _f32),
    ]
    scratch += [pltpu.VMEM((SCH,), jnp.int32)] * SRING
    scratch += [pltpu.VMEM((SCH, HALF), _f32)] * SRING
    scratch += [pltpu.SemaphoreType.DMA] * (2 * SRING)
    fn = pl.kernel(
        functools.partial(_scatter_body, lo),
        out_type=jax.ShapeDtypeStruct((2, 2, N, HALF), _f32),
        mesh=mesh,
        scratch_types=scratch,
        name=f"scatter_lo{lo}",
    )
    return fn(numc, sig, dst)


def _fin_body(h_ref, ah_ref, sums0_ref, sums1_ref, g_ref, b_ref, m_ref,
              v_ref, out_ref):
    sm = sums0_ref[...] + sums1_ref[...]
    num = jnp.concatenate([sm[0, 0], sm[0, 1]], axis=1)
    den = jnp.concatenate([sm[1, 0], sm[1, 1]], axis=1)
    hb = h_ref[...]
    hagg = ah_ref[...] + num / (den + 1e-6)
    hnew = jnp.where(den > 0.0, hagg, hb)
    x = hnew * _f32(1.0 / math.sqrt(N))
    y = (x - m_ref[...]) * lax.rsqrt(v_ref[...] + BN_EPS) * g_ref[...] + b_ref[...]
    out_ref[...] = hb + jnp.maximum(y, 0.0)


def _node_finalize(h, ah, sums0, sums1, g, b, m, v):
    nblk = N // NODE_BLK
    blk = pl.BlockSpec((NODE_BLK, D), lambda i: (i, 0))
    vec = pl.BlockSpec((1, D), lambda i: (0, 0))
    sspec = pl.BlockSpec((2, 2, NODE_BLK, HALF), lambda i: (0, 0, i, 0))
    return pl.pallas_call(
        _fin_body,
        grid=(nblk,),
        in_specs=[blk, blk, sspec, sspec, vec, vec, vec, vec],
        out_specs=blk,
        out_shape=jax.ShapeDtypeStruct((N, D), _f32),
    )(h, ah, sums0, sums1, g.reshape(1, D), b.reshape(1, D),
      m.reshape(1, D), v.reshape(1, D))


def kernel(h, e, edge_index, etype, A_w, A_b, B_w, B_b, C_w, C_b, E_w, E_b,
           weight, w_comp, bn_h_gamma, bn_h_beta, bn_h_mean, bn_h_var,
           bn_e_gamma, bn_e_beta, bn_e_mean, bn_e_var):
    src = edge_index[0]
    dst = edge_index[1]
    ah, bh, eh, hall = _node_proj(h, A_w, A_b, B_w, B_b, E_w, E_b,
                                  weight, w_comp)
    hallf = hall.reshape(R * N, D)
    g0 = _sc_gather(hallf, eh, bh, src, dst, etype, 0)
    g1 = _sc_gather(hallf, eh, bh, src, dst, etype, E2)
    e_new0, numc0, sig0 = _edge_fused(e, *g0, C_w, C_b, bn_e_gamma,
                                      bn_e_beta, bn_e_mean, bn_e_var, 0, None)
    sums0 = _sc_scatter(numc0, sig0, dst, 0)
    e_new, numc1, sig1 = _edge_fused(e, *g1, C_w, C_b, bn_e_gamma,
                                     bn_e_beta, bn_e_mean, bn_e_var, E2,
                                     e_new0)
    sums1 = _sc_scatter(numc1, sig1, dst, E2)
    h_new = _node_finalize(h, ah, sums0, sums1, bn_h_gamma, bn_h_beta,
                           bn_h_mean, bn_h_var)
    return (h_new, e_new)


# speedup vs baseline: 2.8835x; 1.0119x over previous
import functools
import math

import jax
import jax.numpy as jnp
from jax import lax
from jax.experimental import pallas as pl
from jax.experimental.pallas import tpu as pltpu
from jax.experimental.pallas import tpu_sc as plsc

N = 10000
E = 160000
D = 256
HALF = 128
R = 8
NB = 4
BN_EPS = 1e-5

NODE_BLK = 400
EDGE_BLK = 800
E2 = E // 2
NW = 32
GEPW = 2504
GEPW_LAST = E2 - (NW - 1) * GEPW
SCH = 104
SEPT = E2 // 16
RPT = 624
ZROWS = 104

_f32 = jnp.float32


def _node_proj_body(wc_ref, h_ref, aw_ref, ab_ref, bw_ref, bb_ref, ew_ref,
                    eb_ref, wt_ref, ah_o, bh_o, eh_o, hall_o):
    hb = h_ref[...]
    ah_o[...] = jnp.dot(hb, aw_ref[...], preferred_element_type=_f32) + ab_ref[...]
    bh_o[...] = jnp.dot(hb, bw_ref[...], preferred_element_type=_f32) + bb_ref[...]
    eh_o[...] = jnp.dot(hb, ew_ref[...], preferred_element_type=_f32) + eb_ref[...]
    base = [jnp.dot(hb, wt_ref[b], preferred_element_type=_f32) for b in range(NB)]
    for r in range(R):
        acc = base[0] * wc_ref[0, r * NB]
        for b in range(1, NB):
            acc = acc + base[b] * wc_ref[0, r * NB + b]
        hall_o[r] = acc


def _node_proj(h, A_w, A_b, B_w, B_b, E_w, E_b, weight, w_comp):
    nblk = N // NODE_BLK
    vec_spec = pl.BlockSpec((1, D), lambda i: (0, 0))
    mat_spec = pl.BlockSpec((D, D), lambda i: (0, 0))
    out = pl.pallas_call(
        _node_proj_body,
        grid=(nblk,),
        in_specs=[
            pl.BlockSpec(memory_space=pltpu.SMEM),
            pl.BlockSpec((NODE_BLK, D), lambda i: (i, 0)),
            mat_spec, vec_spec, mat_spec, vec_spec, mat_spec, vec_spec,
            pl.BlockSpec((NB, D, D), lambda i: (0, 0, 0)),
        ],
        out_specs=[
            pl.BlockSpec((NODE_BLK, D), lambda i: (i, 0)),
            pl.BlockSpec((NODE_BLK, D), lambda i: (i, 0)),
            pl.BlockSpec((NODE_BLK, D), lambda i: (i, 0)),
            pl.BlockSpec((R, NODE_BLK, D), lambda i: (0, i, 0)),
        ],
        out_shape=[
            jax.ShapeDtypeStruct((N, D), _f32),
            jax.ShapeDtypeStruct((N, D), _f32),
            jax.ShapeDtypeStruct((N, D), _f32),
            jax.ShapeDtypeStruct((R, N, D), _f32),
        ],
    )(w_comp.reshape(1, R * NB), h, A_w, A_b.reshape(1, D), B_w,
      B_b.reshape(1, D), E_w, E_b.reshape(1, D), weight)
    return out


GCH = 80
GRING = 2
GCHUNKS = 32
GGROUPS = GCHUNKS // GRING


def _gather_body(lo, hall_ref, eh_ref, bh_ref, src_ref, dst_ref, ety_ref,
                 msg_o, ehd_o, bhs_o, *scr):
    bufs = [scr[b * 7:(b + 1) * 7] for b in range(GRING)]
    sems = [scr[7 * GRING + b * 7: 7 * GRING + (b + 1) * 7]
            for b in range(GRING)]
    wid = lax.axis_index("s") * 2 + lax.axis_index("c")
    base = wid * GEPW
    epw_w = jnp.where(wid == NW - 1, GEPW_LAST, GEPW)

    def off_of(j):
        return base + jnp.minimum(j * GCH, epw_w - GCH)

    def issue_idx(j, b):
        i_src, i_ety, i_dst = bufs[b][0], bufs[b][1], bufs[b][2]
        semi = sems[b][0]
        off = lo + off_of(j)
        pltpu.async_copy(src_ref.at[pl.ds(off, GCH)], i_src, semi)
        pltpu.async_copy(ety_ref.at[pl.ds(off, GCH)], i_ety, semi)
        pltpu.async_copy(dst_ref.at[pl.ds(off, GCH)], i_dst, semi)

    def wait_idx(j, b):
        i_src, i_ety, i_dst = bufs[b][0], bufs[b][1], bufs[b][2]
        semi = sems[b][0]
        off = lo + off_of(j)
        pltpu.make_async_copy(src_ref.at[pl.ds(off, GCH)], i_src, semi).wait()
        pltpu.make_async_copy(ety_ref.at[pl.ds(off, GCH)], i_ety, semi).wait()
        pltpu.make_async_copy(dst_ref.at[pl.ds(off, GCH)], i_dst, semi).wait()

    def step(j, b, wait_wb):
        i_src, i_ety, i_dst, i_msg, rmsg, rehd, rbhs = bufs[b]
        semi, semg1, semg2, semg3, semw1, semw2, semw3 = sems[b]
        wait_idx(j, b)
        for k in range(GCH // 16):
            sl = pl.ds(k * 16, 16)
            i_msg[sl] = i_ety[sl] * N + i_src[sl]
        if wait_wb:
            po = pl.ds(off_of(j - GRING), GCH)
            pltpu.make_async_copy(rmsg, msg_o.at[po], semw1).wait()
            pltpu.make_async_copy(rehd, ehd_o.at[po], semw2).wait()
            pltpu.make_async_copy(rbhs, bhs_o.at[po], semw3).wait()
        off = off_of(j)
        d1 = pltpu.async_copy(hall_ref.at[i_msg], rmsg, semg1)
        d2 = pltpu.async_copy(eh_ref.at[i_dst], rehd, semg2)
        d3 = pltpu.async_copy(bh_ref.at[i_src], rbhs, semg3)
        d1.wait()
        pltpu.async_copy(rmsg, msg_o.at[pl.ds(off, GCH)], semw1)
        d2.wait()
        pltpu.async_copy(rehd, ehd_o.at[pl.ds(off, GCH)], semw2)
        d3.wait()
        pltpu.async_copy(rbhs, bhs_o.at[pl.ds(off, GCH)], semw3)
        issue_idx(j + GRING, b)

    for b in range(GRING):
        issue_idx(b, b)
    for b in range(GRING):
        step(b, b, False)

    def group(g, carry):
        for b in range(GRING):
            step(g * GRING + b, b, True)
        return carry
    lax.fori_loop(1, GGROUPS, group, 0)

    for b in range(GRING):
        j = GCHUNKS - GRING + b
        po = pl.ds(off_of(j), GCH)
        pltpu.make_async_copy(bufs[b][4], msg_o.at[po], sems[b][4]).wait()
        pltpu.make_async_copy(bufs[b][5], ehd_o.at[po], sems[b][5]).wait()
        pltpu.make_async_copy(bufs[b][6], bhs_o.at[po], sems[b][6]).wait()
        wait_idx(j + GRING, b)


def _sc_gather(hall_flat, eh, bh, src, dst, ety, lo):
    mesh = plsc.VectorSubcoreMesh(core_axis_name="c", subcore_axis_name="s")
    scratch = []
    for _ in range(GRING):
        scratch += [pltpu.VMEM((GCH,), jnp.int32)] * 4
        scratch += [pltpu.VMEM((GCH, D), _f32)] * 3
    scratch += [pltpu.SemaphoreType.DMA] * (7 * GRING)
    fn = pl.kernel(
        functools.partial(_gather_body, lo),
        out_type=[
            jax.ShapeDtypeStruct((E2, D), _f32),
            jax.ShapeDtypeStruct((E2, D), _f32),
            jax.ShapeDtypeStruct((E2, D), _f32),
        ],
        mesh=mesh,
        scratch_types=scratch,
        name=f"gather_lo{lo}",
    )
    return fn(hall_flat, eh, bh, src, dst, ety)


def _edge_body(e_ref, msg_ref, ehd_ref, bhs_ref, cw_ref, cb_ref,
               g_ref, b_ref, m_ref, v_ref, enew_o, numc_o, sig_o):
    eb = e_ref[...]
    ce = jnp.dot(eb, cw_ref[...], preferred_element_type=_f32) + cb_ref[...]
    eij = ce + msg_ref[...] + ehd_ref[...]
    sig = jax.nn.sigmoid(eij)
    numc = sig * bhs_ref[...]
    numc_o[0] = numc[:, :HALF]
    numc_o[1] = numc[:, HALF:]
    sig_o[0] = sig[:, :HALF]
    sig_o[1] = sig[:, HALF:]
    x = eij * _f32(1.0 / math.sqrt(E))
    y = (x - m_ref[...]) * lax.rsqrt(v_ref[...] + BN_EPS) * g_ref[...] + b_ref[...]
    enew_o[...] = eb + jnp.maximum(y, 0.0)


def _edge_body_alias(e_ref, msg_ref, ehd_ref, bhs_ref, cw_ref, cb_ref,
                     g_ref, b_ref, m_ref, v_ref, prev_ref,
                     enew_o, numc_o, sig_o):
    _edge_body(e_ref, msg_ref, ehd_ref, bhs_ref, cw_ref, cb_ref,
               g_ref, b_ref, m_ref, v_ref, enew_o, numc_o, sig_o)


def _edge_fused(e, msg, ehd, bhs, C_w, C_b, g, b, m, v, lo, prev):
    eblk = E2 // EDGE_BLK
    boff = lo // EDGE_BLK
    eblk_spec = pl.BlockSpec((EDGE_BLK, D), lambda i: (i + boff, 0))
    blk = pl.BlockSpec((EDGE_BLK, D), lambda i: (i, 0))
    vec = pl.BlockSpec((1, D), lambda i: (0, 0))
    half = pl.BlockSpec((2, EDGE_BLK, HALF), lambda i: (0, i, 0))
    in_specs = [eblk_spec, blk, blk, blk,
                pl.BlockSpec((D, D), lambda i: (0, 0)),
                vec, vec, vec, vec, vec]
    args = [e, msg, ehd, bhs, C_w, C_b.reshape(1, D), g.reshape(1, D),
            b.reshape(1, D), m.reshape(1, D), v.reshape(1, D)]
    body = _edge_body
    io_alias = {}
    if prev is not None:
        in_specs = in_specs + [pl.BlockSpec(memory_space=pl.ANY)]
        args = args + [prev]
        body = _edge_body_alias
        io_alias = {10: 0}
    return pl.pallas_call(
        body,
        grid=(eblk,),
        in_specs=in_specs,
        out_specs=[eblk_spec, half, half],
        out_shape=[
            jax.ShapeDtypeStruct((E, D), _f32),
            jax.ShapeDtypeStruct((2, E2, HALF), _f32),
            jax.ShapeDtypeStruct((2, E2, HALF), _f32),
        ],
        input_output_aliases=io_alias,
    )(*args)


SCHUNKS = 48
SRING = 2
SGROUPS = SCHUNKS // SRING


def _scatter_body(lo, numc_ref, sig_ref, dst_ref, out_ref,
                  acc, zbuf, idx_t, vals_t, *scr):
    idxs = [scr[b] for b in range(SRING)]
    vals = [scr[SRING + b] for b in range(SRING)]
    semin = [scr[2 * SRING + b] for b in range(SRING)]
    semadd = [scr[3 * SRING + b] for b in range(SRING)]
    c = lax.axis_index("c")
    s = lax.axis_index("s")
    last = s == 15

    def zrow(k, carry):
        i = k // 8
        j = (k % 8) * 16
        zbuf[i, pl.ds(j, 16)] = jnp.zeros((16,), _f32)
        return carry
    lax.fori_loop(0, ZROWS * 8, zrow, 0)

    ebase = s * SEPT
    rbase = s * RPT
    for q, inref in ((0, numc_ref), (1, sig_ref)):
        def zero(k, carry):
            pltpu.sync_copy(zbuf, acc.at[pl.ds(rbase + k * ZROWS, ZROWS)])
            return carry
        lax.fori_loop(0, RPT // ZROWS, zero, 0)
        pl.when(last)(lambda: pltpu.sync_copy(
            zbuf.at[pl.ds(0, 16)], acc.at[pl.ds(16 * RPT, 16)]))
        plsc.subcore_barrier()

        def off_of(j):
            return ebase + jnp.minimum(j * SCH, (SCHUNKS - 1) * SCH)

        def issue_loads(j, b):
            off = off_of(j)
            pltpu.async_copy(dst_ref.at[pl.ds(lo + off, SCH)], idxs[b],
                             semin[b])
            pltpu.async_copy(inref.at[c, pl.ds(off, SCH), :], vals[b],
                             semin[b])

        def wait_loads(j, b):
            off = off_of(j)
            pltpu.make_async_copy(
                dst_ref.at[pl.ds(lo + off, SCH)], idxs[b], semin[b]).wait()
            pltpu.make_async_copy(
                inref.at[c, pl.ds(off, SCH), :], vals[b], semin[b]).wait()

        for b in range(SRING):
            issue_loads(b, b)

        def group(g, carry):
            for b in range(SRING):
                j = g * SRING + b
                wait_loads(j, b)
                pltpu.async_copy(vals[b], acc.at[idxs[b]], semadd[b],
                                 add=True).wait()
                issue_loads(j + SRING, b)
            return carry
        lax.fori_loop(0, SGROUPS, group, 0)
        for b in range(SRING):
            wait_loads(SCHUNKS + b, b)

        toff = ebase + SCHUNKS * SCH
        pltpu.sync_copy(dst_ref.at[pl.ds(lo + toff, 8)], idx_t)
        pltpu.sync_copy(inref.at[c, pl.ds(toff, 8), :], vals_t)
        pltpu.sync_copy(vals_t, acc.at[idx_t], add=True)
        plsc.subcore_barrier()

        pltpu.sync_copy(acc.at[pl.ds(rbase, RPT)],
                        out_ref.at[q, c, pl.ds(rbase, RPT), :])
        pl.when(last)(lambda: pltpu.sync_copy(
            acc.at[pl.ds(16 * RPT, 16)],
            out_ref.at[q, c, pl.ds(16 * RPT, 16), :]))


def _sc_scatter(numc, sig, dst, lo):
    mesh = plsc.VectorSubcoreMesh(core_axis_name="c", subcore_axis_name="s")
    scratch = [
        pltpu.VMEM_SHARED((N, HALF), _f32),
        pltpu.VMEM((ZROWS, HALF), _f32),
        pltpu.VMEM((8,), jnp.int32),
        pltpu.VMEM((8, HALF), _f32),
    ]
    scratch += [pltpu.VMEM((SCH,), jnp.int32)] * SRING
    scratch += [pltpu.VMEM((SCH, HALF), _f32)] * SRING
    scratch += [pltpu.SemaphoreType.DMA] * (2 * SRING)
    fn = pl.kernel(
        functools.partial(_scatter_body, lo),
        out_type=jax.ShapeDtypeStruct((2, 2, N, HALF), _f32),
        mesh=mesh,
        scratch_types=scratch,
        name=f"scatter_lo{lo}",
    )
    return fn(numc, sig, dst)


def _fin_body(h_ref, ah_ref, sums0_ref, sums1_ref, g_ref, b_ref, m_ref,
              v_ref, out_ref):
    sm = sums0_ref[...] + sums1_ref[...]
    num = jnp.concatenate([sm[0, 0], sm[0, 1]], axis=1)
    den = jnp.concatenate([sm[1, 0], sm[1, 1]], axis=1)
    hb = h_ref[...]
    hagg = ah_ref[...] + num / (den + 1e-6)
    hnew = jnp.where(den > 0.0, hagg, hb)
    x = hnew * _f32(1.0 / math.sqrt(N))
    y = (x - m_ref[...]) * lax.rsqrt(v_ref[...] + BN_EPS) * g_ref[...] + b_ref[...]
    out_ref[...] = hb + jnp.maximum(y, 0.0)


def _node_finalize(h, ah, sums0, sums1, g, b, m, v):
    nblk = N // NODE_BLK
    blk = pl.BlockSpec((NODE_BLK, D), lambda i: (i, 0))
    vec = pl.BlockSpec((1, D), lambda i: (0, 0))
    sspec = pl.BlockSpec((2, 2, NODE_BLK, HALF), lambda i: (0, 0, i, 0))
    return pl.pallas_call(
        _fin_body,
        grid=(nblk,),
        in_specs=[blk, blk, sspec, sspec, vec, vec, vec, vec],
        out_specs=blk,
        out_shape=jax.ShapeDtypeStruct((N, D), _f32),
    )(h, ah, sums0, sums1, g.reshape(1, D), b.reshape(1, D),
      m.reshape(1, D), v.reshape(1, D))


def kernel(h, e, edge_index, etype, A_w, A_b, B_w, B_b, C_w, C_b, E_w, E_b,
           weight, w_comp, bn_h_gamma, bn_h_beta, bn_h_mean, bn_h_var,
           bn_e_gamma, bn_e_beta, bn_e_mean, bn_e_var):
    src = edge_index[0]
    dst = edge_index[1]
    ah, bh, eh, hall = _node_proj(h, A_w, A_b, B_w, B_b, E_w, E_b,
                                  weight, w_comp)
    hallf = hall.reshape(R * N, D)
    g0 = _sc_gather(hallf, eh, bh, src, dst, etype, 0)
    g1 = _sc_gather(hallf, eh, bh, src, dst, etype, E2)
    e_new0, numc0, sig0 = _edge_fused(e, *g0, C_w, C_b, bn_e_gamma,
                                      bn_e_beta, bn_e_mean, bn_e_var, 0, None)
    sums0 = _sc_scatter(numc0, sig0, dst, 0)
    e_new, numc1, sig1 = _edge_fused(e, *g1, C_w, C_b, bn_e_gamma,
                                     bn_e_beta, bn_e_mean, bn_e_var, E2,
                                     e_new0)
    sums1 = _sc_scatter(numc1, sig1, dst, E2)
    h_new = _node_finalize(h, ah, sums0, sums1, bn_h_gamma, bn_h_beta,
                           bn_h_mean, bn_h_var)
    return (h_new, e_new)
